# baseline (speedup 1.0000x reference)
import jax
import jax.numpy as jnp
from jax.experimental import pallas as pl
from jax.experimental.pallas import tpu as pltpu
from jax.experimental.pallas import tpu_sc as plsc

_B = 1024
_D = 128
_P = 100000
_TEMP = 0.05
_P_BLK = 2000
_NBLK = _P // _P_BLK
_GATHER_W = 128


def _sc_gather(storage, indices):
    idx = indices.astype(jnp.int32).reshape(1, _B)
    mesh = plsc.VectorSubcoreMesh(core_axis_name="core", subcore_axis_name="subcore")

    @pl.kernel(
        out_type=jax.ShapeDtypeStruct((_B, _D), storage.dtype),
        mesh=mesh,
    )
    def gather_kernel(st_hbm, i_hbm, o_hbm):
        def body(i_vmem, o_vmem):
            pltpu.sync_copy(st_hbm.at[i_vmem.at[0]], o_vmem)

        pltpu.emit_pipeline(
            body,
            grid=(_B // _GATHER_W,),
            in_specs=[pl.BlockSpec((1, _GATHER_W), lambda i: (0, i))],
            out_specs=[pl.BlockSpec((_GATHER_W, _D), lambda i: (i, 0))],
            core_axis_name="subcore",
            dimension_semantics=(pltpu.PARALLEL,),
        )(i_hbm, o_hbm)

    return gather_kernel(storage, idx)


_CHUNK = 500
_NCHUNK = _P_BLK // _CHUNK


def _lse_body(x_ref, st_ref, g_ref, loss_ref, sc_buf, m_acc, s_acc):
    j = pl.program_id(0)

    @pl.when(j == 0)
    def _init():
        m_acc[...] = jnp.full((1, _B), -jnp.inf, jnp.float32)
        s_acc[...] = jnp.zeros((1, _B), jnp.float32)
        sc_buf[1] = jnp.full((_P_BLK, _B), -1e30, jnp.float32)

    st = st_ref[...].astype(jnp.bfloat16)
    sc_buf[j % 2] = jax.lax.dot_general(
        st, x_ref[...], (((1,), (0,)), ((), ())),
        preferred_element_type=jnp.float32,
    )

    buf = (j + 1) % 2
    m = m_acc[...]
    s = s_acc[...]
    for c in range(_NCHUNK):
        sc = sc_buf[buf, pl.ds(c * _CHUNK, _CHUNK), :]
        m_new = jnp.maximum(m, jnp.max(sc, axis=0, keepdims=True))
        s = s * jnp.exp(m - m_new) + jnp.sum(
            jnp.exp(sc - m_new), axis=0, keepdims=True
        )
        m = m_new
    m_acc[...] = m
    s_acc[...] = s

    @pl.when(j == _NBLK)
    def _finish():
        picked = jnp.sum(
            x_ref[...].astype(jnp.float32) * g_ref[...].astype(jnp.float32),
            axis=0,
            keepdims=True,
        )
        lse = jnp.log(s_acc[...]) + m_acc[...]
        loss_ref[...] = jnp.sum(lse - picked, keepdims=True) * (1.0 / _B)


def _fused_loss(x_t, storage, g_t, interpret=False):
    return pl.pallas_call(
        _lse_body,
        grid=(_NBLK + 1,),
        in_specs=[
            pl.BlockSpec((_D, _B), lambda j: (0, 0)),
            pl.BlockSpec((_P_BLK, _D), lambda j: (jnp.minimum(j, _NBLK - 1), 0)),
            pl.BlockSpec((_D, _B), lambda j: (0, 0)),
        ],
        out_specs=pl.BlockSpec((1, 1), lambda j: (0, 0)),
        out_shape=jax.ShapeDtypeStruct((1, 1), jnp.float32),
        scratch_shapes=[
            pltpu.VMEM((2, _P_BLK, _B), jnp.float32),
            pltpu.VMEM((1, _B), jnp.float32),
            pltpu.VMEM((1, _B), jnp.float32),
        ],
        compiler_params=pltpu.CompilerParams(
            dimension_semantics=("arbitrary",),
        ),
        interpret=interpret,
    )(x_t, storage, g_t)


def kernel(input_features, camids, proxy_labels, abs_proxy_labels, storage):
    del camids, proxy_labels
    g = _sc_gather(storage, abs_proxy_labels)
    x_t = (input_features * (1.0 / _TEMP)).T.astype(jnp.bfloat16)
    g_t = g.T.astype(jnp.bfloat16)
    loss = _fused_loss(x_t, storage, g_t)
    return loss[0, 0]


# speedup vs baseline: 1.2143x; 1.2143x over previous
import jax
import jax.numpy as jnp
from jax.experimental import pallas as pl
from jax.experimental.pallas import tpu as pltpu
from jax.experimental.pallas import tpu_sc as plsc

_B = 1024
_D = 128
_P = 100000
_TEMP = 0.05
_P_BLK = 2000
_NBLK = _P // _P_BLK
_GATHER_W = 128


def _sc_gather(storage, indices):
    idx = indices.astype(jnp.int32).reshape(1, _B)
    mesh = plsc.VectorSubcoreMesh(core_axis_name="core", subcore_axis_name="subcore")

    @pl.kernel(
        out_type=jax.ShapeDtypeStruct((_B, _D), storage.dtype),
        mesh=mesh,
    )
    def gather_kernel(st_hbm, i_hbm, o_hbm):
        def body(i_vmem, o_vmem):
            pltpu.sync_copy(st_hbm.at[i_vmem.at[0]], o_vmem)

        pltpu.emit_pipeline(
            body,
            grid=(_B // _GATHER_W,),
            in_specs=[pl.BlockSpec((1, _GATHER_W), lambda i: (0, i))],
            out_specs=[pl.BlockSpec((_GATHER_W, _D), lambda i: (i, 0))],
            core_axis_name="subcore",
            dimension_semantics=(pltpu.PARALLEL,),
        )(i_hbm, o_hbm)

    return gather_kernel(storage, idx)


_CHUNK = 500
_NCHUNK = _P_BLK // _CHUNK
_SUP = 2 * _P_BLK
_NSUP = _P // _SUP


def _online_reduce(buf_ref, gate, m_acc, s_acc):
    m = m_acc[...]
    s = s_acc[...]
    for c in range(_NCHUNK):
        sc = buf_ref[pl.ds(c * _CHUNK, _CHUNK), :]
        m_new = jnp.maximum(m, jnp.max(sc, axis=0, keepdims=True))
        s = s * jnp.exp(m - m_new) + gate * jnp.sum(
            jnp.exp(sc - m_new), axis=0, keepdims=True
        )
        m = m_new
    m_acc[...] = m
    s_acc[...] = s


def _lse_body(x_ref, st_ref, g_ref, loss_ref, buf_a, buf_b, m_acc, s_acc):
    j = pl.program_id(0)

    @pl.when(j == 0)
    def _init():
        m_acc[...] = jnp.full((1, _B), -jnp.inf, jnp.float32)
        s_acc[...] = jnp.zeros((1, _B), jnp.float32)
        buf_b[...] = jnp.full((_P_BLK, _B), -1e30, jnp.float32)

    st = st_ref[...].astype(jnp.bfloat16)
    x = x_ref[...]
    dims = (((1,), (0,)), ((), ()))
    buf_a[...] = jax.lax.dot_general(
        st[: _P_BLK], x, dims, preferred_element_type=jnp.float32
    )
    _online_reduce(buf_b, 1.0, m_acc, s_acc)
    buf_b[...] = jax.lax.dot_general(
        st[_P_BLK :], x, dims, preferred_element_type=jnp.float32
    )
    gate_a = jnp.where(j < _NSUP, 1.0, 0.0)
    _online_reduce(buf_a, gate_a, m_acc, s_acc)

    @pl.when(j == _NSUP)
    def _finish():
        picked = jnp.sum(
            x_ref[...].astype(jnp.float32) * g_ref[...].astype(jnp.float32),
            axis=0,
            keepdims=True,
        )
        lse = jnp.log(s_acc[...]) + m_acc[...]
        loss_ref[...] = jnp.sum(lse - picked, keepdims=True) * (1.0 / _B)


def _fused_loss(x_t, storage, g_t, interpret=False):
    return pl.pallas_call(
        _lse_body,
        grid=(_NSUP + 1,),
        in_specs=[
            pl.BlockSpec((_D, _B), lambda j: (0, 0)),
            pl.BlockSpec((_SUP, _D), lambda j: (jnp.minimum(j, _NSUP - 1), 0)),
            pl.BlockSpec((_D, _B), lambda j: (0, 0)),
        ],
        out_specs=pl.BlockSpec((1, 1), lambda j: (0, 0)),
        out_shape=jax.ShapeDtypeStruct((1, 1), jnp.float32),
        scratch_shapes=[
            pltpu.VMEM((_P_BLK, _B), jnp.float32),
            pltpu.VMEM((_P_BLK, _B), jnp.float32),
            pltpu.VMEM((1, _B), jnp.float32),
            pltpu.VMEM((1, _B), jnp.float32),
        ],
        compiler_params=pltpu.CompilerParams(
            dimension_semantics=("arbitrary",),
        ),
        interpret=interpret,
    )(x_t, storage, g_t)


def kernel(input_features, camids, proxy_labels, abs_proxy_labels, storage):
    del camids, proxy_labels
    g = _sc_gather(storage, abs_proxy_labels)
    x_t = (input_features * (1.0 / _TEMP)).T.astype(jnp.bfloat16)
    g_t = g.T.astype(jnp.bfloat16)
    loss = _fused_loss(x_t, storage, g_t)
    return loss[0, 0]


# speedup vs baseline: 1.3894x; 1.1442x over previous
import jax
import jax.numpy as jnp
from jax.experimental import pallas as pl
from jax.experimental.pallas import tpu as pltpu
from jax.experimental.pallas import tpu_sc as plsc

_B = 1024
_D = 128
_P = 100000
_TEMP = 0.05
_P_BLK = 2000
_NBLK = _P // _P_BLK
_GATHER_W = 128


def _sc_gather(storage, indices):
    idx = indices.astype(jnp.int32).reshape(1, _B)
    mesh = plsc.VectorSubcoreMesh(core_axis_name="core", subcore_axis_name="subcore")

    @pl.kernel(
        out_type=jax.ShapeDtypeStruct((_B, _D), storage.dtype),
        mesh=mesh,
    )
    def gather_kernel(st_hbm, i_hbm, o_hbm):
        def body(i_vmem, o_vmem):
            pltpu.sync_copy(st_hbm.at[i_vmem.at[0]], o_vmem)

        pltpu.emit_pipeline(
            body,
            grid=(_B // _GATHER_W,),
            in_specs=[pl.BlockSpec((1, _GATHER_W), lambda i: (0, i))],
            out_specs=[pl.BlockSpec((_GATHER_W, _D), lambda i: (i, 0))],
            core_axis_name="subcore",
            dimension_semantics=(pltpu.PARALLEL,),
        )(i_hbm, o_hbm)

    return gather_kernel(storage, idx)


_CHUNK = 2000
_NCHUNK = _P_BLK // _CHUNK
_SUP = 2 * _P_BLK
_NSUP = _P // _SUP


def _online_reduce(buf_ref, gate, m_acc, s_acc):
    m = m_acc[...]
    s = s_acc[...]
    for c in range(_NCHUNK):
        sc = buf_ref[pl.ds(c * _CHUNK, _CHUNK), :]
        m_new = jnp.maximum(m, jnp.max(sc, axis=0, keepdims=True))
        s = s * jnp.exp(m - m_new) + gate * jnp.sum(
            jnp.exp(sc - m_new), axis=0, keepdims=True
        )
        m = m_new
    m_acc[...] = m
    s_acc[...] = s


def _lse_body(x_ref, st_ref, g_ref, loss_ref, buf_a, buf_b, m_acc, s_acc):
    j = pl.program_id(0)

    @pl.when(j == 0)
    def _init():
        m_acc[...] = jnp.full((1, _B), -jnp.inf, jnp.float32)
        s_acc[...] = jnp.zeros((1, _B), jnp.float32)
        buf_b[...] = jnp.full((_P_BLK, _B), -1e30, jnp.float32)

    st = st_ref[...].astype(jnp.bfloat16)
    x = x_ref[...]
    dims = (((1,), (0,)), ((), ()))
    buf_a[...] = jax.lax.dot_general(
        st[: _P_BLK], x, dims, preferred_element_type=jnp.float32
    )
    _online_reduce(buf_b, 1.0, m_acc, s_acc)
    buf_b[...] = jax.lax.dot_general(
        st[_P_BLK :], x, dims, preferred_element_type=jnp.float32
    )
    gate_a = jnp.where(j < _NSUP, 1.0, 0.0)
    _online_reduce(buf_a, gate_a, m_acc, s_acc)

    @pl.when(j == _NSUP)
    def _finish():
        picked = jnp.sum(
            x_ref[...].astype(jnp.float32) * g_ref[...].astype(jnp.float32),
            axis=0,
            keepdims=True,
        )
        lse = jnp.log(s_acc[...]) + m_acc[...]
        loss_ref[...] = jnp.sum(lse - picked, keepdims=True) * (1.0 / _B)


def _fused_loss(x_t, storage, g_t, interpret=False):
    return pl.pallas_call(
        _lse_body,
        grid=(_NSUP + 1,),
        in_specs=[
            pl.BlockSpec((_D, _B), lambda j: (0, 0)),
            pl.BlockSpec((_SUP, _D), lambda j: (jnp.minimum(j, _NSUP - 1), 0)),
            pl.BlockSpec((_D, _B), lambda j: (0, 0)),
        ],
        out_specs=pl.BlockSpec((1, 1), lambda j: (0, 0)),
        out_shape=jax.ShapeDtypeStruct((1, 1), jnp.float32),
        scratch_shapes=[
            pltpu.VMEM((_P_BLK, _B), jnp.float32),
            pltpu.VMEM((_P_BLK, _B), jnp.float32),
            pltpu.VMEM((1, _B), jnp.float32),
            pltpu.VMEM((1, _B), jnp.float32),
        ],
        compiler_params=pltpu.CompilerParams(
            dimension_semantics=("arbitrary",),
        ),
        interpret=interpret,
    )(x_t, storage, g_t)


def kernel(input_features, camids, proxy_labels, abs_proxy_labels, storage):
    del camids, proxy_labels
    g = _sc_gather(storage, abs_proxy_labels)
    x_t = (input_features * (1.0 / _TEMP)).T.astype(jnp.bfloat16)
    g_t = g.T.astype(jnp.bfloat16)
    loss = _fused_loss(x_t, storage, g_t)
    return loss[0, 0]
